# baseline (speedup 1.0000x reference)
import functools

import jax
import jax.numpy as jnp
from jax import lax
from jax.experimental import pallas as pl
from jax.experimental.pallas import tpu as pltpu
from jax.experimental.pallas import tpu_sc as plsc

DIM = 64
PAD = 0
LANES = 16

N_CORES = 2
N_SUBCORES = 16
NW = N_CORES * N_SUBCORES

CHUNK = 128
SUPER = 5
SROWS = SUPER * CHUNK
NROWS = 7


def _emb_body(idx_hbm, table_hbm, out_hbm, idx_v, rows_v, gsem, *, n_super, bc):
    cid = lax.axis_index("c")
    sid = lax.axis_index("s")
    wid = sid * N_CORES + cid
    n_chunks = n_super * SUPER
    c_base = wid * n_chunks
    l0 = c_base // bc

    pltpu.sync_copy(idx_hbm.at[pl.ds(l0, NROWS)], idx_v)

    zeros = jnp.zeros((LANES,), jnp.float32)

    def chunk_coords(c):
        return c // bc - l0, (c % bc) * CHUNK

    def fire(sup, buf):
        for j in range(SUPER):
            lr, b0 = chunk_coords(c_base + sup * SUPER + j)
            pltpu.async_copy(
                table_hbm.at[idx_v.at[lr, pl.ds(b0, CHUNK)]],
                rows_v.at[buf, pl.ds(j * CHUNK, CHUNK)],
                gsem,
            )

    def drain(sup, buf):
        for j in range(SUPER):
            lr, b0 = chunk_coords(c_base + sup * SUPER + j)
            pltpu.make_async_copy(
                table_hbm.at[idx_v.at[lr, pl.ds(b0, CHUNK)]],
                rows_v.at[buf, pl.ds(j * CHUNK, CHUNK)],
                gsem,
            ).wait()

    def process(sup, buf):
        def group_body(g, carry):
            lr, b0 = chunk_coords(c_base + sup * SUPER + g // (CHUNK // LANES))
            off = (g % (CHUNK // LANES)) * LANES
            iv = idx_v[lr, pl.ds(b0 + off, LANES)]
            m = iv == PAD
            npad = plsc.all_reduce_population_count(m)[0]

            @pl.when(npad > 0)
            def _zero_pad_rows():
                rows16 = g * LANES + lax.iota(jnp.int32, LANES)
                for col in range(DIM):
                    plsc.store_scatter(
                        rows_v.at[buf],
                        [rows16, jnp.full((LANES,), col, jnp.int32)],
                        zeros,
                        mask=m,
                    )

            return carry

        lax.fori_loop(0, SROWS // LANES, group_body, 0)

    def write(sup, buf):
        for j in range(SUPER):
            lr, b0 = chunk_coords(c_base + sup * SUPER + j)
            pltpu.sync_copy(
                rows_v.at[buf, pl.ds(j * CHUNK, CHUNK)],
                out_hbm.at[lr + l0, pl.ds(b0, CHUNK)],
            )

    fire(0, 0)

    def pipe_body(t, carry):
        s0 = 2 * t
        s1 = 2 * t + 1

        drain(s0, 0)
        fire(s1, 1)
        process(s0, 0)
        write(s0, 0)

        drain(s1, 1)

        @pl.when(s1 + 1 < n_super)
        def _fire_next():
            fire(s1 + 1, 0)

        process(s1, 1)
        write(s1, 1)
        return carry

    lax.fori_loop(0, n_super // 2, pipe_body, 0)


def kernel(src_input, emb_table):
    L, B, _ = src_input.shape
    total = L * B
    assert B % CHUNK == 0
    bc = B // CHUNK
    assert total % (NW * SROWS) == 0 and (total // (NW * SROWS)) % 2 == 0
    n_super = total // (NW * SROWS)
    assert (n_super * SUPER) % 2 == 0 or NROWS * bc >= n_super * SUPER + bc - 1

    idx = src_input[..., 0]

    mesh = plsc.VectorSubcoreMesh(core_axis_name="c", subcore_axis_name="s")
    run = functools.partial(
        pl.kernel,
        mesh=mesh,
        out_type=jax.ShapeDtypeStruct((L, B, DIM), jnp.float32),
        scratch_types=[
            pltpu.VMEM((NROWS, B), jnp.int32),
            pltpu.VMEM((2, SROWS, DIM), jnp.float32),
            pltpu.SemaphoreType.DMA,
        ],
        compiler_params=pltpu.CompilerParams(
            needs_layout_passes=False, use_tc_tiling_on_sc=False
        ),
    )(functools.partial(_emb_body, n_super=n_super, bc=bc))

    return run(idx, emb_table)
